# Initial kernel scaffold; baseline (speedup 1.0000x reference)
import jax
import jax.numpy as jnp
from jax.experimental import pallas as pl


def kernel(x, tables):
    raise NotImplementedError("write your pallas kernel here")


# speedup vs baseline: 4.3602x; 4.3602x over previous
import functools

import jax
import jax.numpy as jnp
import numpy as np
from jax import lax
from jax.experimental import pallas as pl
from jax.experimental.pallas import tpu as pltpu
from jax.experimental.pallas import tpu_sc as plsc

_N_LEVELS = 16
_F = 2
_LOG2_H = 19
_H = 2 ** _LOG2_H
_B = 1048576
_L = 16

_NC = 2
_NS = 16
_NW = _NC * _NS
_PTS_PER_W = _B // _NW
_C = 512
_N_CHUNKS = _PTS_PER_W // _C

_P1 = np.int32(-1640531535)
_P2 = np.int32(805459861)
_MASK = np.int32(_H - 1)


def _hash_gather_kernel(x_hbm, tbl_hbm, out_hbm, xbuf, idxbuf, rowsbuf, sem):
    wid = lax.axis_index("s") * _NC + lax.axis_index("c")
    iota = lax.iota(jnp.int32, _L)
    iota3 = iota * 3
    iota16 = iota * _L

    def chunk_body(ci, carry):
        base = wid * _PTS_PER_W + ci * _C
        pltpu.sync_copy(x_hbm.at[pl.ds(base * 3, 3 * _C)], xbuf)

        def vec_body(i, carry2):
            xoff = iota3 + i * (3 * _L)
            x0 = plsc.load_gather(xbuf, [xoff])
            x1 = plsc.load_gather(xbuf, [xoff + 1])
            x2 = plsc.load_gather(xbuf, [xoff + 2])
            off_base = iota16 + i * (_L * _N_LEVELS)
            for lvl in range(_N_LEVELS):
                scale = np.float32(2.0 ** lvl)
                c0 = (x0 * scale).astype(jnp.int32)
                c1 = (x1 * scale).astype(jnp.int32)
                c2 = (x2 * scale).astype(jnp.int32)
                h = (c0 + c1 * _P1 + c2 * _P2) & _MASK
                flat = h | np.int32(lvl << _LOG2_H)
                plsc.store_scatter(idxbuf, [off_base + lvl], flat)
            return carry2

        lax.fori_loop(0, _C // _L, vec_body, 0, unroll=False)

        pltpu.async_copy(tbl_hbm.at[idxbuf], rowsbuf, sem).wait()
        pltpu.sync_copy(
            rowsbuf, out_hbm.at[pl.ds(base * _N_LEVELS, _C * _N_LEVELS)]
        )
        return carry

    lax.fori_loop(0, _N_CHUNKS, chunk_body, 0, unroll=False)


@functools.partial(jax.jit, static_argnames=())
def kernel(x, tables):
    x_flat = x.reshape(3 * _B)
    tbl_flat = tables.reshape(_N_LEVELS * _H, _F)
    mesh = plsc.VectorSubcoreMesh(core_axis_name="c", subcore_axis_name="s")
    run = functools.partial(
        pl.kernel,
        out_type=jax.ShapeDtypeStruct((_B * _N_LEVELS, _F), jnp.float32),
        mesh=mesh,
        compiler_params=pltpu.CompilerParams(
            needs_layout_passes=False, use_tc_tiling_on_sc=False
        ),
        scratch_types=[
            pltpu.VMEM((3 * _C,), jnp.float32),
            pltpu.VMEM((_N_LEVELS * _C,), jnp.int32),
            pltpu.VMEM((_N_LEVELS * _C, _F), jnp.float32),
            pltpu.SemaphoreType.DMA,
        ],
    )(_hash_gather_kernel)
    out = run(x_flat, tbl_flat)
    return out.reshape(_B, _N_LEVELS * _F)


# speedup vs baseline: 7.5128x; 1.7230x over previous
import functools

import jax
import jax.numpy as jnp
import numpy as np
from jax import lax
from jax.experimental import pallas as pl
from jax.experimental.pallas import tpu as pltpu
from jax.experimental.pallas import tpu_sc as plsc

_N_LEVELS = 16
_F = 2
_LOG2_H = 19
_H = 2 ** _LOG2_H
_B = 1048576
_L = 16

_NC = 2
_NS = 16
_NW = _NC * _NS
_PTS_PER_W = _B // _NW
_C = 512
_N_CHUNKS = _PTS_PER_W // _C
_R = _N_LEVELS * _C

_P1 = np.int32(-1640531535)
_P2 = np.int32(805459861)
_MASK = np.int32(_H - 1)


def _hash_gather_kernel(x_hbm, tbl_hbm, out_hbm, xbuf, idxbuf, idxrows,
                        rowsbuf, outtile, sem):
    wid = lax.axis_index("s") * _NC + lax.axis_index("c")
    iota = lax.iota(jnp.int32, _L)
    iota16 = iota * _L
    col0 = iota * 0
    col1 = col0 + 1
    col2 = col0 + 2
    half = lax.shift_right_logical(iota, 1)
    parity = iota & 1

    def chunk_body(ci, carry):
        base = wid * _PTS_PER_W + ci * _C
        pltpu.sync_copy(x_hbm.at[pl.ds(base, _C), :], xbuf.at[:, pl.ds(0, 3)])

        def vec_body(i, carry2):
            rows = iota + i * _L
            x0 = plsc.load_gather(xbuf, [rows, col0])
            x1 = plsc.load_gather(xbuf, [rows, col1])
            x2 = plsc.load_gather(xbuf, [rows, col2])
            off_base = iota16 + i * (_L * _N_LEVELS)
            for lvl in range(_N_LEVELS):
                scale = np.float32(2.0 ** lvl)
                c0 = (x0 * scale).astype(jnp.int32)
                c1 = (x1 * scale).astype(jnp.int32)
                c2 = (x2 * scale).astype(jnp.int32)
                h = (c0 + c1 * _P1 + c2 * _P2) & _MASK
                flat = h | np.int32(lvl << _LOG2_H)
                plsc.store_scatter(idxbuf, [off_base + lvl], flat)
            return carry2

        lax.fori_loop(0, _C // _L, vec_body, 0, unroll=2)

        def shift_body(k, carry2):
            v = idxbuf[pl.ds(k * _L, _L)]
            idxrows[pl.ds(k * _L, _L)] = lax.shift_right_logical(v, 2)
            return carry2

        lax.fori_loop(0, _R // _L, shift_body, 0, unroll=4)

        pltpu.async_copy(tbl_hbm.at[idxrows], rowsbuf, sem).wait()

        def pack_body(r, carry3):
            for cc in range(8):
                grow = half + (r * 8 + cc) * 8
                e = plsc.load_gather(idxbuf, [grow])
                gcol = (e & 3) * 2 + parity
                v = plsc.load_gather(rowsbuf, [grow, gcol])
                outtile[r, pl.ds(cc * _L, _L)] = v
            return carry3

        lax.fori_loop(0, _C // 4, pack_body, 0, unroll=2)

        pltpu.sync_copy(outtile, out_hbm.at[pl.ds(base // 4, _C // 4), :])
        return carry

    lax.fori_loop(0, _N_CHUNKS, chunk_body, 0, unroll=False)


def kernel(x, tables):
    tbl_wide = tables.reshape(_N_LEVELS * _H * _F // 8, 8)
    mesh = plsc.VectorSubcoreMesh(core_axis_name="c", subcore_axis_name="s")
    run = functools.partial(
        pl.kernel,
        out_type=jax.ShapeDtypeStruct((_B // 4, 128), jnp.float32),
        mesh=mesh,
        compiler_params=pltpu.CompilerParams(
            needs_layout_passes=False, use_tc_tiling_on_sc=False
        ),
        scratch_types=[
            pltpu.VMEM((_C, 8), jnp.float32),
            pltpu.VMEM((_R,), jnp.int32),
            pltpu.VMEM((_R,), jnp.int32),
            pltpu.VMEM((_R, 8), jnp.float32),
            pltpu.VMEM((_C // 4, 128), jnp.float32),
            pltpu.SemaphoreType.DMA,
        ],
    )(_hash_gather_kernel)
    out = run(x, tbl_wide)
    return out.reshape(_B, _N_LEVELS * _F)


# speedup vs baseline: 7.7109x; 1.0264x over previous
import functools

import jax
import jax.numpy as jnp
import numpy as np
from jax import lax
from jax.experimental import pallas as pl
from jax.experimental.pallas import tpu as pltpu
from jax.experimental.pallas import tpu_sc as plsc

_N_LEVELS = 16
_F = 2
_LOG2_H = 19
_H = 2 ** _LOG2_H
_B = 1048576
_L = 16

_NC = 2
_NS = 16
_NW = _NC * _NS
_PTS_PER_W = _B // _NW
_C = 512
_N_CHUNKS = _PTS_PER_W // _C
_R = _N_LEVELS * _C
_G = _C // 128

_P1 = np.int32(-1640531535)
_P2 = np.int32(805459861)
_MASK = np.int32(_H - 1)


def _hash_gather_kernel(x_hbm, tbl_hbm, out_hbm, xbuf, idxbuf, idxrows,
                        rowsbuf, outtile, sem):
    wid = lax.axis_index("s") * _NC + lax.axis_index("c")
    iota = lax.iota(jnp.int32, _L)
    iota16 = iota * _L
    col0 = iota * 0
    col1 = col0 + 1
    col2 = col0 + 2

    def chunk_body(ci, carry):
        base = wid * _PTS_PER_W + ci * _C
        pltpu.sync_copy(x_hbm.at[pl.ds(base, _C), :], xbuf.at[:, pl.ds(0, 3)])

        def vec_body(i, carry2):
            rows = iota + i * _L
            x0 = plsc.load_gather(xbuf, [rows, col0])
            x1 = plsc.load_gather(xbuf, [rows, col1])
            x2 = plsc.load_gather(xbuf, [rows, col2])
            off_base = iota16 + i * (_L * _N_LEVELS)
            for lvl in range(_N_LEVELS):
                scale = np.float32(2.0 ** lvl)
                c0 = (x0 * scale).astype(jnp.int32)
                c1 = (x1 * scale).astype(jnp.int32)
                c2 = (x2 * scale).astype(jnp.int32)
                h = (c0 + c1 * _P1 + c2 * _P2) & _MASK
                flat = h | np.int32(lvl << _LOG2_H)
                plsc.store_scatter(idxbuf, [off_base + lvl], flat)
            return carry2

        lax.fori_loop(0, _C // _L, vec_body, 0, unroll=2)

        def shift_body(k, carry2):
            v = idxbuf[pl.ds(k * _L, _L)]
            idxrows[pl.ds(k * _L, _L)] = lax.shift_right_logical(v, 2)
            return carry2

        lax.fori_loop(0, _R // _L, shift_body, 0, unroll=4)

        pltpu.async_copy(tbl_hbm.at[idxrows], rowsbuf, sem).wait()

        def pack_body(g, carry3):
            gb = g * (128 * _N_LEVELS)
            for rt in range(4):
                for fr in range(8):
                    f = 8 * rt + fr
                    lvl = f >> 1
                    par = f & 1
                    for k in range(8):
                        dvec = iota16 + (gb + k * (_L * _N_LEVELS) + lvl)
                        e = plsc.load_gather(idxbuf, [dvec])
                        gcol = (e & 3) * 2 + par
                        v = plsc.load_gather(rowsbuf, [dvec, gcol])
                        outtile[rt, g, pl.ds(fr * 128 + k * _L, _L)] = v
            return carry3

        lax.fori_loop(0, _G, pack_body, 0, unroll=False)

        for rt in range(4):
            pltpu.sync_copy(
                outtile.at[rt],
                out_hbm.at[rt, pl.ds(base // 128, _G), :],
            )
        return carry

    lax.fori_loop(0, _N_CHUNKS, chunk_body, 0, unroll=False)


def kernel(x, tables):
    tbl_wide = tables.reshape(_N_LEVELS * _H * _F // 8, 8)
    mesh = plsc.VectorSubcoreMesh(core_axis_name="c", subcore_axis_name="s")
    run = functools.partial(
        pl.kernel,
        out_type=jax.ShapeDtypeStruct((4, _B // 128, 1024), jnp.float32),
        mesh=mesh,
        compiler_params=pltpu.CompilerParams(
            needs_layout_passes=False, use_tc_tiling_on_sc=False
        ),
        scratch_types=[
            pltpu.VMEM((_C, 8), jnp.float32),
            pltpu.VMEM((_R,), jnp.int32),
            pltpu.VMEM((_R,), jnp.int32),
            pltpu.VMEM((_R, 8), jnp.float32),
            pltpu.VMEM((4, _G, 1024), jnp.float32),
            pltpu.SemaphoreType.DMA,
        ],
    )(_hash_gather_kernel)
    out = run(x, tbl_wide)
    return (
        out.reshape(4, _B // 128, 8, 128)
        .transpose(1, 3, 0, 2)
        .reshape(_B, _N_LEVELS * _F)
    )


# speedup vs baseline: 18.7325x; 2.4293x over previous
import functools

import jax
import jax.numpy as jnp
import numpy as np
from jax import lax
from jax.experimental import pallas as pl
from jax.experimental.pallas import tpu as pltpu
from jax.experimental.pallas import tpu_sc as plsc

_N_LEVELS = 16
_F = 2
_LOG2_H = 19
_H = 2 ** _LOG2_H
_B = 1048576
_L = 16

_NC = 2
_NS = 16
_NW = _NC * _NS
_PTS_PER_W = _B // _NW
_C = 512
_N_CHUNKS = _PTS_PER_W // _C
_R = _N_LEVELS * _C
_G = _C // 128

_NBLK = _H // 128
_BLK_PER_W = _N_LEVELS * _NBLK // _NW
_BI = 16

_P1 = np.int32(-1640531535)
_P2 = np.int32(805459861)
_MASK = np.int32(_H - 1)


def _interleave_kernel(w_hbm, flat_hbm, wbuf, obuf):
    wid = lax.axis_index("s") * _NC + lax.axis_index("c")
    lvl = lax.shift_right_logical(wid, 1)
    blk_base = (wid & 1) * _BLK_PER_W
    iota = lax.iota(jnp.int32, _L)
    iota2 = iota * 2

    def iter_body(it, carry):
        b0 = blk_base + it * _BI
        pltpu.sync_copy(w_hbm.at[lvl, pl.ds(b0, _BI), :, :], wbuf)
        for bi in range(_BI):
            for k in range(8):
                f0 = wbuf[bi, 0, pl.ds(k * _L, _L)]
                f1 = wbuf[bi, 1, pl.ds(k * _L, _L)]
                off = iota2 + (bi * 256 + k * 32)
                plsc.store_scatter(
                    obuf, [lax.shift_right_logical(off, 3), off & 7], f0
                )
                off1 = off + 1
                plsc.store_scatter(
                    obuf, [lax.shift_right_logical(off1, 3), off1 & 7], f1
                )
        row0 = lvl * (_H // 4) + b0 * 32
        pltpu.sync_copy(obuf, flat_hbm.at[pl.ds(row0, _BI * 32), :])
        return carry

    lax.fori_loop(0, _BLK_PER_W // _BI, iter_body, 0, unroll=False)


def _hash_gather_kernel(x_hbm, tbl_hbm, out_hbm, xbuf, idxbuf, idxrows,
                        rowsbuf, outtile, sem):
    wid = lax.axis_index("s") * _NC + lax.axis_index("c")
    iota = lax.iota(jnp.int32, _L)
    iota16 = iota * _L
    col0 = iota * 0
    col1 = col0 + 1
    col2 = col0 + 2

    def chunk_body(ci, carry):
        base = wid * _PTS_PER_W + ci * _C
        pltpu.sync_copy(x_hbm.at[pl.ds(base, _C), :], xbuf.at[:, pl.ds(0, 3)])

        def vec_body(i, carry2):
            rows = iota + i * _L
            x0 = plsc.load_gather(xbuf, [rows, col0])
            x1 = plsc.load_gather(xbuf, [rows, col1])
            x2 = plsc.load_gather(xbuf, [rows, col2])
            off_base = iota16 + i * (_L * _N_LEVELS)
            for lvl in range(_N_LEVELS):
                scale = np.float32(2.0 ** lvl)
                c0 = (x0 * scale).astype(jnp.int32)
                c1 = (x1 * scale).astype(jnp.int32)
                c2 = (x2 * scale).astype(jnp.int32)
                h = (c0 + c1 * _P1 + c2 * _P2) & _MASK
                flat = h | np.int32(lvl << _LOG2_H)
                plsc.store_scatter(idxbuf, [off_base + lvl], flat)
            return carry2

        lax.fori_loop(0, _C // _L, vec_body, 0, unroll=2)

        def shift_body(k, carry2):
            v = idxbuf[pl.ds(k * _L, _L)]
            idxrows[pl.ds(k * _L, _L)] = lax.shift_right_logical(v, 2)
            return carry2

        lax.fori_loop(0, _R // _L, shift_body, 0, unroll=4)

        pltpu.async_copy(tbl_hbm.at[idxrows], rowsbuf, sem).wait()

        def pack_body(g, carry3):
            gb = g * (128 * _N_LEVELS)
            for rt in range(4):
                for fr in range(8):
                    f = 8 * rt + fr
                    lvl = f >> 1
                    par = f & 1
                    for k in range(8):
                        dvec = iota16 + (gb + k * (_L * _N_LEVELS) + lvl)
                        e = plsc.load_gather(idxbuf, [dvec])
                        gcol = (e & 3) * 2 + par
                        v = plsc.load_gather(rowsbuf, [dvec, gcol])
                        outtile[rt, g, pl.ds(fr * 128 + k * _L, _L)] = v
            return carry3

        lax.fori_loop(0, _G, pack_body, 0, unroll=False)

        for rt in range(4):
            pltpu.sync_copy(
                outtile.at[rt],
                out_hbm.at[rt, pl.ds(base // 128, _G), :],
            )
        return carry

    lax.fori_loop(0, _N_CHUNKS, chunk_body, 0, unroll=False)


def kernel(x, tables):
    mesh = plsc.VectorSubcoreMesh(core_axis_name="c", subcore_axis_name="s")
    params = pltpu.CompilerParams(
        needs_layout_passes=False, use_tc_tiling_on_sc=False
    )

    w = tables.reshape(_N_LEVELS, _NBLK, 128, _F).transpose(0, 1, 3, 2)
    interleave = functools.partial(
        pl.kernel,
        out_type=jax.ShapeDtypeStruct((_N_LEVELS * _H * _F // 8, 8),
                                      jnp.float32),
        mesh=mesh,
        compiler_params=params,
        scratch_types=[
            pltpu.VMEM((_BI, _F, 128), jnp.float32),
            pltpu.VMEM((_BI * 32, 8), jnp.float32),
        ],
    )(_interleave_kernel)
    tbl_flat = interleave(w)

    run = functools.partial(
        pl.kernel,
        out_type=jax.ShapeDtypeStruct((4, _B // 128, 1024), jnp.float32),
        mesh=mesh,
        compiler_params=params,
        scratch_types=[
            pltpu.VMEM((_C, 8), jnp.float32),
            pltpu.VMEM((_R,), jnp.int32),
            pltpu.VMEM((_R,), jnp.int32),
            pltpu.VMEM((_R, 8), jnp.float32),
            pltpu.VMEM((4, _G, 1024), jnp.float32),
            pltpu.SemaphoreType.DMA,
        ],
    )(_hash_gather_kernel)
    out = run(x, tbl_flat)
    return (
        out.reshape(4, _B // 128, 8, 128)
        .transpose(1, 3, 0, 2)
        .reshape(_B, _N_LEVELS * _F)
    )


# speedup vs baseline: 22.2894x; 1.1899x over previous
import functools

import jax
import jax.numpy as jnp
import numpy as np
from jax import lax
from jax.experimental import pallas as pl
from jax.experimental.pallas import tpu as pltpu
from jax.experimental.pallas import tpu_sc as plsc

_N_LEVELS = 16
_F = 2
_LOG2_H = 19
_H = 2 ** _LOG2_H
_B = 1048576
_L = 16

_NC = 2
_NS = 16
_NW = _NC * _NS
_PTS_PER_W = _B // _NW
_C = 512
_N_CHUNKS = _PTS_PER_W // _C
_R = _N_LEVELS * _C
_G = _C // 128

_NBLK = _H // 128
_BLK_PER_W = _N_LEVELS * _NBLK // _NW
_BI = 16

_P1 = np.int32(-1640531535)
_P2 = np.int32(805459861)
_MASK = np.int32(_H - 1)


def _interleave_kernel(w_hbm, flat_hbm, wbuf, obuf):
    wid = lax.axis_index("s") * _NC + lax.axis_index("c")
    lvl = lax.shift_right_logical(wid, 1)
    blk_base = (wid & 1) * _BLK_PER_W
    iota = lax.iota(jnp.int32, _L)
    iota2 = iota * 2

    def iter_body(it, carry):
        b0 = blk_base + it * _BI
        pltpu.sync_copy(w_hbm.at[lvl, pl.ds(b0, _BI), :, :], wbuf)
        for bi in range(_BI):
            for k in range(8):
                f0 = wbuf[bi, 0, pl.ds(k * _L, _L)]
                f1 = wbuf[bi, 1, pl.ds(k * _L, _L)]
                off = iota2 + (bi * 256 + k * 32)
                plsc.store_scatter(
                    obuf, [lax.shift_right_logical(off, 3), off & 7], f0
                )
                off1 = off + 1
                plsc.store_scatter(
                    obuf, [lax.shift_right_logical(off1, 3), off1 & 7], f1
                )
        row0 = lvl * (_H // 4) + b0 * 32
        pltpu.sync_copy(obuf, flat_hbm.at[pl.ds(row0, _BI * 32), :])
        return carry

    lax.fori_loop(0, _BLK_PER_W // _BI, iter_body, 0, unroll=False)


def _hash_gather_kernel(x_hbm, tbl_hbm, out_hbm, xbuf, idxbuf, idxrows,
                        rowsbuf, outtile, sem):
    wid = lax.axis_index("s") * _NC + lax.axis_index("c")
    iota = lax.iota(jnp.int32, _L)
    iota16 = iota * _L

    def chunk_body(ci, carry):
        base = wid * _PTS_PER_W + ci * _C
        pltpu.sync_copy(x_hbm.at[pl.ds(base // 128, _G), :, :], xbuf)

        def vec_body(i, carry2):
            blk = lax.shift_right_logical(i, 3)
            w16 = (i & 7) * _L
            x0 = xbuf[blk, 0, pl.ds(w16, _L)]
            x1 = xbuf[blk, 1, pl.ds(w16, _L)]
            x2 = xbuf[blk, 2, pl.ds(w16, _L)]
            off_base = iota16 + i * (_L * _N_LEVELS)
            pt = iota + i * _L
            for lvl in range(_N_LEVELS):
                scale = np.float32(2.0 ** lvl)
                c0 = (x0 * scale).astype(jnp.int32)
                c1 = (x1 * scale).astype(jnp.int32)
                c2 = (x2 * scale).astype(jnp.int32)
                h = (c0 + c1 * _P1 + c2 * _P2) & _MASK
                flat = h | np.int32(lvl << _LOG2_H)
                plsc.store_scatter(idxbuf, [off_base + lvl], flat)
                plsc.store_scatter(idxrows, [pt + lvl * _C],
                                   lax.shift_right_logical(flat, 2))
            return carry2

        lax.fori_loop(0, _C // _L, vec_body, 0, unroll=2)

        copies = []
        for lvl in range(_N_LEVELS):
            copies.append(pltpu.async_copy(
                tbl_hbm.at[idxrows.at[pl.ds(lvl * _C, _C)]],
                rowsbuf.at[pl.ds(lvl * _C, _C), :],
                sem,
            ))
        for cp in copies:
            cp.wait()

        def pack_body(g, carry3):
            gp = g * 128
            for rt in range(4):
                for fr in range(8):
                    f = 8 * rt + fr
                    lvl = f >> 1
                    par = f & 1
                    for k in range(8):
                        dvec = iota + (lvl * _C + gp + k * _L)
                        e = plsc.load_gather(
                            idxbuf,
                            [iota16 + ((gp + k * _L) * _N_LEVELS + lvl)])
                        gcol = (e & 3) * 2 + par
                        v = plsc.load_gather(rowsbuf, [dvec, gcol])
                        outtile[rt, g, pl.ds(fr * 128 + k * _L, _L)] = v
            return carry3

        lax.fori_loop(0, _G, pack_body, 0, unroll=False)

        for rt in range(4):
            pltpu.sync_copy(
                outtile.at[rt],
                out_hbm.at[rt, pl.ds(base // 128, _G), :],
            )
        return carry

    lax.fori_loop(0, _N_CHUNKS, chunk_body, 0, unroll=False)


def kernel(x, tables):
    mesh = plsc.VectorSubcoreMesh(core_axis_name="c", subcore_axis_name="s")
    params = pltpu.CompilerParams(
        needs_layout_passes=False, use_tc_tiling_on_sc=False
    )

    w = tables.reshape(_N_LEVELS, _NBLK, 128, _F).transpose(0, 1, 3, 2)
    interleave = functools.partial(
        pl.kernel,
        out_type=jax.ShapeDtypeStruct((_N_LEVELS * _H * _F // 8, 8),
                                      jnp.float32),
        mesh=mesh,
        compiler_params=params,
        scratch_types=[
            pltpu.VMEM((_BI, _F, 128), jnp.float32),
            pltpu.VMEM((_BI * 32, 8), jnp.float32),
        ],
    )(_interleave_kernel)
    tbl_flat = interleave(w)

    xv = jnp.concatenate(
        [
            x.T.reshape(3, _B // 128, 128),
            jnp.zeros((5, _B // 128, 128), jnp.float32),
        ],
        axis=0,
    ).transpose(1, 0, 2)
    run = functools.partial(
        pl.kernel,
        out_type=jax.ShapeDtypeStruct((4, _B // 128, 1024), jnp.float32),
        mesh=mesh,
        compiler_params=params,
        scratch_types=[
            pltpu.VMEM((_G, 8, 128), jnp.float32),
            pltpu.VMEM((_R,), jnp.int32),
            pltpu.VMEM((_R,), jnp.int32),
            pltpu.VMEM((_R, 8), jnp.float32),
            pltpu.VMEM((4, _G, 1024), jnp.float32),
            pltpu.SemaphoreType.DMA,
        ],
    )(_hash_gather_kernel)
    out = run(xv, tbl_flat)
    return (
        out.reshape(4, _B // 128, 8, 128)
        .transpose(1, 3, 0, 2)
        .reshape(_B, _N_LEVELS * _F)
    )


# speedup vs baseline: 65.7564x; 2.9501x over previous
import functools

import jax
import jax.numpy as jnp
import numpy as np
from jax import lax
from jax.experimental import pallas as pl
from jax.experimental.pallas import tpu as pltpu
from jax.experimental.pallas import tpu_sc as plsc

_N_LEVELS = 16
_F = 2
_LOG2_H = 19
_H = 2 ** _LOG2_H
_B = 1048576
_L = 16

_NC = 2
_NS = 16
_NW = _NC * _NS
_PTS_PER_W = _B // _NW
_C = 512
_N_CHUNKS = _PTS_PER_W // _C
_R = _N_LEVELS * _C
_G = _C // 128

_NBLK = _H // 128
_BLK_PER_W = _N_LEVELS * _NBLK // _NW
_BI = 16

_P1 = np.int32(-1640531535)
_P2 = np.int32(805459861)
_MASK = np.int32(_H - 1)

_LUT_LEVELS = 5
_NGL = _N_LEVELS - _LUT_LEVELS
_CB = [0, 1, 9, 73, 585]
_NCELLS = 4681
_NSETUP = 4688


def _interleave_kernel(w_hbm, flat_hbm, wbuf, obuf):
    wid = lax.axis_index("s") * _NC + lax.axis_index("c")
    lvl = lax.shift_right_logical(wid, 1)
    blk_base = (wid & 1) * _BLK_PER_W
    iota = lax.iota(jnp.int32, _L)
    iota2 = iota * 2

    def iter_body(it, carry):
        b0 = blk_base + it * _BI
        pltpu.sync_copy(w_hbm.at[lvl, pl.ds(b0, _BI), :, :], wbuf)
        for bi in range(_BI):
            for k in range(8):
                f0 = wbuf[bi, 0, pl.ds(k * _L, _L)]
                f1 = wbuf[bi, 1, pl.ds(k * _L, _L)]
                off = iota2 + (bi * 256 + k * 32)
                plsc.store_scatter(
                    obuf, [lax.shift_right_logical(off, 3), off & 7], f0
                )
                off1 = off + 1
                plsc.store_scatter(
                    obuf, [lax.shift_right_logical(off1, 3), off1 & 7], f1
                )
        row0 = lvl * (_H // 4) + b0 * 32
        pltpu.sync_copy(obuf, flat_hbm.at[pl.ds(row0, _BI * 32), :])
        return carry

    lax.fori_loop(0, _BLK_PER_W // _BI, iter_body, 0, unroll=False)


def _hash_gather_kernel(x_hbm, tbl_hbm, out_hbm, xbuf, idxbuf, idxrows,
                        rowsbuf, outtile, lutbuf, sem):
    wid = lax.axis_index("s") * _NC + lax.axis_index("c")
    iota = lax.iota(jnp.int32, _L)
    iota16 = iota * _L
    iota2 = iota * 2

    idxrows[pl.ds(_NSETUP - _L, _L)] = iota * 0
    for lvl in range(_LUT_LEVELS):
        n_cells = 1 << (3 * lvl)
        for j in range((n_cells + _L - 1) // _L):
            n = iota + j * _L
            m = np.int32((1 << lvl) - 1)
            c0 = n & m
            c1 = lax.shift_right_logical(n, lvl) & m
            c2 = lax.shift_right_logical(n, 2 * lvl) & m
            h = (c0 + c1 * _P1 + c2 * _P2) & _MASK
            e = h | np.int32(lvl << _LOG2_H)
            pos = n + _CB[lvl]
            plsc.store_scatter(idxbuf, [pos], e)
            plsc.store_scatter(idxrows, [pos], lax.shift_right_logical(e, 2))
    pltpu.async_copy(
        tbl_hbm.at[idxrows.at[pl.ds(0, _NSETUP)]],
        rowsbuf.at[pl.ds(0, _NSETUP), :],
        sem,
    ).wait()
    for j in range((_NCELLS + _L - 1) // _L):
        g = iota + j * _L
        e = idxbuf[pl.ds(j * _L, _L)]
        ecol = (e & 3) * 2
        v0 = plsc.load_gather(rowsbuf, [g, ecol])
        v1 = plsc.load_gather(rowsbuf, [g, ecol + 1])
        o0 = iota2 + j * (2 * _L)
        plsc.store_scatter(
            lutbuf, [lax.shift_right_logical(o0, 3), o0 & 7], v0)
        o1 = o0 + 1
        plsc.store_scatter(
            lutbuf, [lax.shift_right_logical(o1, 3), o1 & 7], v1)

    def chunk_body(ci, carry):
        base = wid * _PTS_PER_W + ci * _C
        pltpu.sync_copy(x_hbm.at[pl.ds(base // 128, _G), :, :], xbuf)

        def vec_body(i, carry2):
            blk = lax.shift_right_logical(i, 3)
            w16 = (i & 7) * _L
            x0 = xbuf[blk, 0, pl.ds(w16, _L)]
            x1 = xbuf[blk, 1, pl.ds(w16, _L)]
            x2 = xbuf[blk, 2, pl.ds(w16, _L)]
            off_base = iota16 + i * (_L * _N_LEVELS)
            pt = iota + i * _L
            for lvl in range(_N_LEVELS):
                scale = np.float32(2.0 ** lvl)
                c0 = (x0 * scale).astype(jnp.int32)
                c1 = (x1 * scale).astype(jnp.int32)
                c2 = (x2 * scale).astype(jnp.int32)
                if lvl < _LUT_LEVELS:
                    cell = (c0 + c1 * np.int32(1 << lvl)
                            + c2 * np.int32(1 << (2 * lvl)))
                    epos = cell * 2 + np.int32(2 * _CB[lvl])
                    plsc.store_scatter(idxbuf, [off_base + lvl], epos)
                else:
                    h = (c0 + c1 * _P1 + c2 * _P2) & _MASK
                    flat = h | np.int32(lvl << _LOG2_H)
                    plsc.store_scatter(idxbuf, [off_base + lvl], flat)
                    plsc.store_scatter(
                        idxrows, [pt + (lvl - _LUT_LEVELS) * _C],
                        lax.shift_right_logical(flat, 2))
            return carry2

        lax.fori_loop(0, _C // _L, vec_body, 0, unroll=2)

        copies = []
        for gl in range(_N_LEVELS - _LUT_LEVELS):
            copies.append(pltpu.async_copy(
                tbl_hbm.at[idxrows.at[pl.ds(gl * _C, _C)]],
                rowsbuf.at[pl.ds(gl * _C, _C), :],
                sem,
            ))
        for cp in copies:
            cp.wait()

        def pack_body(g, carry3):
            gp = g * 128
            for rt in range(4):
                for fr in range(8):
                    f = 8 * rt + fr
                    lvl = f >> 1
                    par = f & 1
                    for k in range(8):
                        e = plsc.load_gather(
                            idxbuf,
                            [iota16 + ((gp + k * _L) * _N_LEVELS + lvl)])
                        if lvl < _LUT_LEVELS:
                            o = e + par
                            v = plsc.load_gather(
                                lutbuf,
                                [lax.shift_right_logical(o, 3), o & 7])
                        else:
                            dvec = iota + (
                                (lvl - _LUT_LEVELS) * _C + gp + k * _L)
                            gcol = (e & 3) * 2 + par
                            v = plsc.load_gather(rowsbuf, [dvec, gcol])
                        outtile[rt, g, pl.ds(fr * 128 + k * _L, _L)] = v
            return carry3

        lax.fori_loop(0, _G, pack_body, 0, unroll=False)

        for rt in range(4):
            pltpu.sync_copy(
                outtile.at[rt],
                out_hbm.at[rt, pl.ds(base // 128, _G), :],
            )
        return carry

    lax.fori_loop(0, _N_CHUNKS, chunk_body, 0, unroll=False)


def kernel(x, tables):
    mesh = plsc.VectorSubcoreMesh(core_axis_name="c", subcore_axis_name="s")
    params = pltpu.CompilerParams(
        needs_layout_passes=False, use_tc_tiling_on_sc=False
    )

    w = tables.reshape(_N_LEVELS, _NBLK, 128, _F).transpose(0, 1, 3, 2)
    interleave = functools.partial(
        pl.kernel,
        out_type=jax.ShapeDtypeStruct((_N_LEVELS * _H * _F // 8, 8),
                                      jnp.float32),
        mesh=mesh,
        compiler_params=params,
        scratch_types=[
            pltpu.VMEM((_BI, _F, 128), jnp.float32),
            pltpu.VMEM((_BI * 32, 8), jnp.float32),
        ],
    )(_interleave_kernel)
    tbl_flat = interleave(w)

    xv = jnp.concatenate(
        [
            x.T.reshape(3, _B // 128, 128),
            jnp.zeros((5, _B // 128, 128), jnp.float32),
        ],
        axis=0,
    ).transpose(1, 0, 2)
    run = functools.partial(
        pl.kernel,
        out_type=jax.ShapeDtypeStruct((4, _B // 128, 1024), jnp.float32),
        mesh=mesh,
        compiler_params=params,
        scratch_types=[
            pltpu.VMEM((_G, 8, 128), jnp.float32),
            pltpu.VMEM((_N_LEVELS * _C,), jnp.int32),
            pltpu.VMEM((_NGL * _C,), jnp.int32),
            pltpu.VMEM((_NGL * _C, 8), jnp.float32),
            pltpu.VMEM((4, _G, 1024), jnp.float32),
            pltpu.VMEM((1184, 8), jnp.float32),
            pltpu.SemaphoreType.DMA,
        ],
    )(_hash_gather_kernel)
    out = run(xv, tbl_flat)
    return (
        out.reshape(4, _B // 128, 8, 128)
        .transpose(1, 3, 0, 2)
        .reshape(_B, _N_LEVELS * _F)
    )


# speedup vs baseline: 76.9546x; 1.1703x over previous
import functools

import jax
import jax.numpy as jnp
import numpy as np
from jax import lax
from jax.experimental import pallas as pl
from jax.experimental.pallas import tpu as pltpu
from jax.experimental.pallas import tpu_sc as plsc

_N_LEVELS = 16
_F = 2
_LOG2_H = 19
_H = 2 ** _LOG2_H
_B = 1048576
_L = 16

_NC = 2
_NS = 16
_NW = _NC * _NS
_PTS_PER_W = _B // _NW
_C = 256
_N_CHUNKS = _PTS_PER_W // _C
_G = _C // 128

_NBLK = _H // 128
_BLK_PER_W = _N_LEVELS * _NBLK // _NW
_BI = 16

_P1 = np.int32(-1640531535)
_P2 = np.int32(805459861)
_MASK = np.int32(_H - 1)

_LUT_LEVELS = 5
_NGL = _N_LEVELS - _LUT_LEVELS
_CB = [0, 1, 9, 73, 585]
_N03 = 585
_N03P = 592
_NROWS = _NGL * _C
_L4SPLIT = _NROWS
_L4REST = 4096 - _L4SPLIT
_L4RESTOFF = 1536


def _interleave_kernel(w_hbm, flat_hbm, wbuf, obuf):
    wid = lax.axis_index("s") * _NC + lax.axis_index("c")
    lvl = lax.shift_right_logical(wid, 1)
    blk_base = (wid & 1) * _BLK_PER_W
    iota = lax.iota(jnp.int32, _L)
    iota2 = iota * 2

    def iter_body(it, carry):
        b0 = blk_base + it * _BI
        pltpu.sync_copy(w_hbm.at[lvl, pl.ds(b0, _BI), :, :], wbuf)
        for bi in range(_BI):
            for k in range(8):
                f0 = wbuf[bi, 0, pl.ds(k * _L, _L)]
                f1 = wbuf[bi, 1, pl.ds(k * _L, _L)]
                off = iota2 + (bi * 256 + k * 32)
                plsc.store_scatter(
                    obuf, [lax.shift_right_logical(off, 3), off & 7], f0
                )
                off1 = off + 1
                plsc.store_scatter(
                    obuf, [lax.shift_right_logical(off1, 3), off1 & 7], f1
                )
        row0 = lvl * (_H // 4) + b0 * 32
        pltpu.sync_copy(obuf, flat_hbm.at[pl.ds(row0, _BI * 32), :])
        return carry

    lax.fori_loop(0, _BLK_PER_W // _BI, iter_body, 0, unroll=False)


def _hash_gather_kernel(x_hbm, tbl_hbm, out_hbm, xbuf, idxbufA, idxrowsA,
                        rowsbufA, idxbufB, idxrowsB, rowsbufB, outtile,
                        lutbuf, semA, semB):
    wid = lax.axis_index("s") * _NC + lax.axis_index("c")
    iota = lax.iota(jnp.int32, _L)
    iota16 = iota * _L
    iota2 = iota * 2

    idxrowsA[pl.ds(_N03P - _L, _L)] = iota * 0

    def _cell_hash(n, lvl):
        m = np.int32((1 << lvl) - 1)
        c0 = n & m
        c1 = lax.shift_right_logical(n, lvl) & m
        c2 = lax.shift_right_logical(n, 2 * lvl) & m
        h = (c0 + c1 * _P1 + c2 * _P2) & _MASK
        e = h | np.int32(lvl << _LOG2_H)
        return e, lax.shift_right_logical(e, 2)

    for lvl in range(4):
        n_cells = 1 << (3 * lvl)

        def _setup03(j, carry, lvl=lvl):
            n = iota + j * _L
            e, r = _cell_hash(n, lvl)
            pos = n + _CB[lvl]
            plsc.store_scatter(idxbufA, [pos], e)
            plsc.store_scatter(idxrowsA, [pos], r)
            return carry

        lax.fori_loop(0, (n_cells + _L - 1) // _L, _setup03, 0,
                      unroll=False)

    def _setup4b(j, carry):
        n = iota + j * _L
        e, r = _cell_hash(n, 4)
        plsc.store_scatter(idxbufB, [n], e)
        plsc.store_scatter(idxrowsB, [n], r)
        return carry

    lax.fori_loop(0, _L4SPLIT // _L, _setup4b, 0, unroll=False)

    def _setup4a(j, carry):
        n = iota + j * _L
        e, r = _cell_hash(n, 4)
        plsc.store_scatter(idxbufB, [n], e)
        plsc.store_scatter(idxrowsA, [n + (_N03P - _L4SPLIT)], r)
        return carry

    lax.fori_loop(_L4SPLIT // _L, 4096 // _L, _setup4a, 0, unroll=False)
    pltpu.async_copy(
        tbl_hbm.at[idxrowsA.at[pl.ds(0, _N03P)]],
        rowsbufA.at[pl.ds(0, _N03P), :], semA).wait()
    pltpu.async_copy(
        tbl_hbm.at[idxrowsB.at[pl.ds(0, _L4SPLIT)]],
        rowsbufB.at[pl.ds(0, _L4SPLIT), :], semA).wait()
    pltpu.async_copy(
        tbl_hbm.at[idxrowsA.at[pl.ds(_N03P, _L4REST)]],
        rowsbufA.at[pl.ds(_L4RESTOFF, _L4REST), :], semA).wait()

    def _compact(j, ebuf, rbuf, cell0, roff):
        g = iota + j * _L
        e = ebuf[pl.ds(j * _L, _L)]
        ecol = (e & 3) * 2
        v0 = plsc.load_gather(rbuf, [g + roff, ecol])
        v1 = plsc.load_gather(rbuf, [g + roff, ecol + 1])
        o0 = iota2 + (j * _L + cell0) * 2
        plsc.store_scatter(
            lutbuf, [lax.shift_right_logical(o0, 3), o0 & 7], v0)
        o1 = o0 + 1
        plsc.store_scatter(
            lutbuf, [lax.shift_right_logical(o1, 3), o1 & 7], v1)

    def _compact1(j, carry):
        _compact(j, idxbufA, rowsbufA, 0, 0)
        return carry

    lax.fori_loop(0, _N03P // _L, _compact1, 0, unroll=False)

    def _compact2(j, carry):
        _compact(j, idxbufB, rowsbufB, _N03, 0)
        return carry

    lax.fori_loop(0, _L4SPLIT // _L, _compact2, 0, unroll=False)

    def _compact3(j, carry):
        _compact(j, idxbufB, rowsbufA, _N03, _L4RESTOFF - _L4SPLIT)
        return carry

    lax.fori_loop(_L4SPLIT // _L, 4096 // _L, _compact3, 0, unroll=False)

    def _stage(ci, idxb, idxr, rowsb, sem):
        base = wid * _PTS_PER_W + ci * _C
        pltpu.sync_copy(x_hbm.at[pl.ds(base // 128, _G), :, :], xbuf)

        def vec_body(i, carry2):
            blk = lax.shift_right_logical(i, 3)
            w16 = (i & 7) * _L
            x0 = xbuf[blk, 0, pl.ds(w16, _L)]
            x1 = xbuf[blk, 1, pl.ds(w16, _L)]
            x2 = xbuf[blk, 2, pl.ds(w16, _L)]
            off_base = iota16 + i * (_L * _N_LEVELS)
            pt = iota + i * _L
            for lvl in range(_N_LEVELS):
                scale = np.float32(2.0 ** lvl)
                c0 = (x0 * scale).astype(jnp.int32)
                c1 = (x1 * scale).astype(jnp.int32)
                c2 = (x2 * scale).astype(jnp.int32)
                if lvl < _LUT_LEVELS:
                    cell = (c0 + c1 * np.int32(1 << lvl)
                            + c2 * np.int32(1 << (2 * lvl)))
                    epos = cell * 2 + np.int32(2 * _CB[lvl])
                    plsc.store_scatter(idxb, [off_base + lvl], epos)
                else:
                    h = (c0 + c1 * _P1 + c2 * _P2) & _MASK
                    flat = h | np.int32(lvl << _LOG2_H)
                    plsc.store_scatter(idxb, [off_base + lvl], flat)
                    plsc.store_scatter(
                        idxr, [pt + (lvl - _LUT_LEVELS) * _C],
                        lax.shift_right_logical(flat, 2))
            return carry2

        lax.fori_loop(0, _C // _L, vec_body, 0, unroll=False)
        for gl in range(_NGL):
            pltpu.async_copy(
                tbl_hbm.at[idxr.at[pl.ds(gl * _C, _C)]],
                rowsb.at[pl.ds(gl * _C, _C), :],
                sem,
            )

    def _pack_out(ci, idxb, rowsb, sem):
        pltpu.make_async_copy(
            tbl_hbm.at[pl.ds(0, _NROWS), :], rowsb, sem).wait()
        base = wid * _PTS_PER_W + ci * _C

        def pack_body(g, carry3):
            gp = g * 128
            for rt in range(4):
                for fr in range(8):
                    f = 8 * rt + fr
                    lvl = f >> 1
                    par = f & 1
                    for k in range(8):
                        e = plsc.load_gather(
                            idxb,
                            [iota16 + ((gp + k * _L) * _N_LEVELS + lvl)])
                        if lvl < _LUT_LEVELS:
                            o = e + par
                            v = plsc.load_gather(
                                lutbuf,
                                [lax.shift_right_logical(o, 3), o & 7])
                        else:
                            dvec = iota + (
                                (lvl - _LUT_LEVELS) * _C + gp + k * _L)
                            gcol = (e & 3) * 2 + par
                            v = plsc.load_gather(rowsb, [dvec, gcol])
                        outtile[rt, g, pl.ds(fr * 128 + k * _L, _L)] = v
            return carry3

        lax.fori_loop(0, _G, pack_body, 0, unroll=False)
        for rt in range(4):
            pltpu.sync_copy(
                outtile.at[rt],
                out_hbm.at[rt, pl.ds(base // 128, _G), :],
            )

    _stage(jnp.int32(0), idxbufA, idxrowsA, rowsbufA, semA)

    def pair_body(ci2, carry):
        c0i = ci2 * 2
        _stage(c0i + 1, idxbufB, idxrowsB, rowsbufB, semB)
        _pack_out(c0i, idxbufA, rowsbufA, semA)
        _stage(c0i + 2, idxbufA, idxrowsA, rowsbufA, semA)
        _pack_out(c0i + 1, idxbufB, rowsbufB, semB)
        return carry

    lax.fori_loop(0, _N_CHUNKS // 2 - 1, pair_body, 0, unroll=False)

    last = jnp.int32(_N_CHUNKS - 1)
    _stage(last, idxbufB, idxrowsB, rowsbufB, semB)
    _pack_out(last - 1, idxbufA, rowsbufA, semA)
    _pack_out(last, idxbufB, rowsbufB, semB)


def kernel(x, tables):
    mesh = plsc.VectorSubcoreMesh(core_axis_name="c", subcore_axis_name="s")
    params = pltpu.CompilerParams(
        needs_layout_passes=False, use_tc_tiling_on_sc=False
    )

    w = tables.reshape(_N_LEVELS, _NBLK, 128, _F).transpose(0, 1, 3, 2)
    interleave = functools.partial(
        pl.kernel,
        out_type=jax.ShapeDtypeStruct((_N_LEVELS * _H * _F // 8, 8),
                                      jnp.float32),
        mesh=mesh,
        compiler_params=params,
        scratch_types=[
            pltpu.VMEM((_BI, _F, 128), jnp.float32),
            pltpu.VMEM((_BI * 32, 8), jnp.float32),
        ],
    )(_interleave_kernel)
    tbl_flat = interleave(w)

    xv = jnp.concatenate(
        [
            x.T.reshape(3, _B // 128, 128),
            jnp.zeros((5, _B // 128, 128), jnp.float32),
        ],
        axis=0,
    ).transpose(1, 0, 2)
    run = functools.partial(
        pl.kernel,
        out_type=jax.ShapeDtypeStruct((4, _B // 128, 1024), jnp.float32),
        mesh=mesh,
        compiler_params=params,
        scratch_types=[
            pltpu.VMEM((_G, 8, 128), jnp.float32),
            pltpu.VMEM((_N_LEVELS * _C,), jnp.int32),
            pltpu.VMEM((_NROWS,), jnp.int32),
            pltpu.VMEM((_NROWS, 8), jnp.float32),
            pltpu.VMEM((_N_LEVELS * _C,), jnp.int32),
            pltpu.VMEM((_NROWS,), jnp.int32),
            pltpu.VMEM((_NROWS, 8), jnp.float32),
            pltpu.VMEM((4, _G, 1024), jnp.float32),
            pltpu.VMEM((1184, 8), jnp.float32),
            pltpu.SemaphoreType.DMA,
            pltpu.SemaphoreType.DMA,
        ],
    )(_hash_gather_kernel)
    out = run(xv, tbl_flat)
    return (
        out.reshape(4, _B // 128, 8, 128)
        .transpose(1, 3, 0, 2)
        .reshape(_B, _N_LEVELS * _F)
    )
